# Initial kernel scaffold; baseline (speedup 1.0000x reference)
import jax
import jax.numpy as jnp
from jax.experimental import pallas as pl


def kernel(x, edge):
    raise NotImplementedError("write your pallas kernel here")


# speedup vs baseline: 1.1069x; 1.1069x over previous
import functools

import jax
import jax.numpy as jnp
from jax import lax
from jax.experimental import pallas as pl
from jax.experimental.pallas import tpu as pltpu
from jax.experimental.pallas import tpu_sc as plsc

N_NODES = 10000
D = 256
E = 160000

NC = 2
NS = 16
LANES = 16

HALF = N_NODES // NC
ACC_R = HALF + 8
SUB = 128
NSUB = 10
CHUNK = SUB * NSUB
NCHUNK = E // CHUNK


def _sc_aggregate(x, src_f, dst_f):
    mesh = plsc.VectorSubcoreMesh(core_axis_name="c", subcore_axis_name="s")

    @functools.partial(
        pl.kernel,
        out_type=jax.ShapeDtypeStruct((N_NODES, D), jnp.float32),
        mesh=mesh,
        compiler_params=pltpu.CompilerParams(use_tc_tiling_on_sc=False),
        scratch_types=[
            pltpu.VMEM((CHUNK,), jnp.int32),
            pltpu.VMEM((CHUNK,), jnp.int32),
            pltpu.VMEM((SUB, LANES), jnp.float32),
            pltpu.VMEM((ACC_R, LANES), jnp.float32),
            pltpu.SemaphoreType.DMA,
        ],
    )
    def agg(x_hbm, src_hbm, dst_hbm, out_hbm, src_v, dst_v, rows_v, acc_v,
            sem):
        h = lax.axis_index("c")
        q = lax.axis_index("s")
        qc = q * LANES

        def zero(r, carry):
            acc_v[r, :] = jnp.zeros((LANES,), jnp.float32)
            return carry

        lax.fori_loop(0, ACC_R, zero, 0)

        qoff = jnp.full((LANES,), q * N_NODES, jnp.int32)

        def super_chunk(k, carry):
            kb = k * CHUNK
            pltpu.sync_copy(src_hbm.at[pl.ds(kb, CHUNK)], src_v)
            pltpu.sync_copy(dst_hbm.at[pl.ds(h * E + kb, CHUNK)], dst_v)

            def offs(i, carry2):
                src_v[pl.ds(i * LANES, LANES)] = (
                    src_v[pl.ds(i * LANES, LANES)] + qoff)
                return carry2

            lax.fori_loop(0, CHUNK // LANES, offs, 0)

            def sub_chunk(m, carry2):
                mb = m * SUB
                pltpu.async_copy(
                    x_hbm.at[src_v.at[pl.ds(mb, SUB)]], rows_v, sem).wait()
                for g in range(SUB // LANES):
                    dvec = dst_v[pl.ds(mb + g * LANES, LANES)]
                    for l in range(LANES):
                        d0 = dvec[l]
                        plsc.addupdate(acc_v.at[d0], rows_v[g * LANES + l, :])
                return carry2

            lax.fori_loop(0, NSUB, sub_chunk, 0)
            return carry

        lax.fori_loop(0, NCHUNK, super_chunk, 0)

        pltpu.sync_copy(
            acc_v.at[pl.ds(0, HALF)],
            out_hbm.at[pl.ds(h * HALF, HALF), pl.ds(qc, LANES)])

    return agg(x, src_f, dst_f)


def _tc_finish(a):
    blk = 1000

    def body(a_ref, o_ref):
        v = jnp.maximum(a_ref[...], 0.0)
        n = jnp.sqrt(jnp.sum(v * v, axis=1, keepdims=True))
        o_ref[...] = v / jnp.maximum(n, 1e-12)

    return pl.pallas_call(
        body,
        grid=(N_NODES // blk,),
        in_specs=[pl.BlockSpec((blk, D), lambda i: (i, 0))],
        out_specs=pl.BlockSpec((blk, D), lambda i: (i, 0)),
        out_shape=jax.ShapeDtypeStruct((N_NODES, D), jnp.float32),
    )(a)


def kernel(x, edge):
    src = edge[0].astype(jnp.int32)
    dst = edge[2].astype(jnp.int32)

    locs = []
    for h in range(NC):
        l = dst - h * HALF
        locs.append(jnp.where((l >= 0) & (l < HALF), l, HALF))
    dst_f = jnp.concatenate(locs, axis=0)

    xf = x.reshape(N_NODES, NS, LANES).transpose(1, 0, 2).reshape(
        NS * N_NODES, LANES)

    agg = _sc_aggregate(xf, src, dst_f)
    return _tc_finish(agg)


# speedup vs baseline: 1.7743x; 1.6029x over previous
import functools

import jax
import jax.numpy as jnp
from jax import lax
from jax.experimental import pallas as pl
from jax.experimental.pallas import tpu as pltpu
from jax.experimental.pallas import tpu_sc as plsc

N_NODES = 10000
D = 256
E = 160000

NC = 2
NS = 16
LANES = 16

HALF = N_NODES // NC
ACC_R = HALF + 8
SUB = 128
NSUB = 10
CHUNK = SUB * NSUB
NCHUNK = E // CHUNK


def _sc_aggregate(x, src_f, dst_f):
    mesh = plsc.VectorSubcoreMesh(core_axis_name="c", subcore_axis_name="s")

    @functools.partial(
        pl.kernel,
        out_type=jax.ShapeDtypeStruct((N_NODES, D), jnp.float32),
        mesh=mesh,
        compiler_params=pltpu.CompilerParams(use_tc_tiling_on_sc=False),
        scratch_types=[
            pltpu.VMEM((2, CHUNK), jnp.int32),
            pltpu.VMEM((2, CHUNK), jnp.int32),
            pltpu.VMEM((2, SUB, LANES), jnp.float32),
            pltpu.VMEM((ACC_R, LANES), jnp.float32),
            pltpu.SemaphoreType.DMA,
            pltpu.SemaphoreType.DMA,
        ],
    )
    def agg(x_hbm, src_hbm, dst_hbm, out_hbm, src_v, dst_v, rows_v, acc_v,
            isem, gsem):
        h = lax.axis_index("c")
        q = lax.axis_index("s")
        qc = q * LANES

        def idx_copies(k, p):
            kb = k * CHUNK
            a = pltpu.make_async_copy(
                src_hbm.at[pl.ds(kb, CHUNK)], src_v.at[p], isem)
            b = pltpu.make_async_copy(
                dst_hbm.at[pl.ds(h * E + kb, CHUNK)], dst_v.at[p], isem)
            return a, b

        def gather_copy(p, m, mp):
            return pltpu.make_async_copy(
                x_hbm.at[src_v.at[p, pl.ds(m * SUB, SUB)]], rows_v.at[mp],
                gsem)

        for cp in idx_copies(0, 0):
            cp.start()

        def zero(r, carry):
            acc_v[r, :] = jnp.zeros((LANES,), jnp.float32)
            return carry

        lax.fori_loop(0, ACC_R, zero, 0)

        qoff = jnp.full((LANES,), q * N_NODES, jnp.int32)

        def super_chunk(k, carry):
            p = lax.rem(k, 2)
            for cp in idx_copies(k, p):
                cp.wait()

            @pl.when(k < NCHUNK - 1)
            def _prefetch_idx():
                for cp in idx_copies(k + 1, 1 - p):
                    cp.start()

            def offs(i, carry2):
                src_v[p, pl.ds(i * LANES, LANES)] = (
                    src_v[p, pl.ds(i * LANES, LANES)] + qoff)
                return carry2

            lax.fori_loop(0, CHUNK // LANES, offs, 0)

            gather_copy(p, 0, 0).start()

            def sub_chunk(m, carry2):
                mp = lax.rem(m, 2)
                gather_copy(p, m, mp).wait()

                @pl.when(m < NSUB - 1)
                def _prefetch_rows():
                    gather_copy(p, m + 1, 1 - mp).start()

                mb = m * SUB
                for g in range(SUB // LANES):
                    dvec = dst_v[p, pl.ds(mb + g * LANES, LANES)]
                    for l in range(LANES):
                        d0 = dvec[l]
                        plsc.addupdate(acc_v.at[d0],
                                       rows_v[mp, g * LANES + l, :])
                return carry2

            lax.fori_loop(0, NSUB, sub_chunk, 0)
            return carry

        lax.fori_loop(0, NCHUNK, super_chunk, 0)

        pltpu.sync_copy(
            acc_v.at[pl.ds(0, HALF)],
            out_hbm.at[pl.ds(h * HALF, HALF), pl.ds(qc, LANES)])

    return agg(x, src_f, dst_f)


def _tc_finish(a):
    blk = 1000

    def body(a_ref, o_ref):
        v = jnp.maximum(a_ref[...], 0.0)
        n = jnp.sqrt(jnp.sum(v * v, axis=1, keepdims=True))
        o_ref[...] = v / jnp.maximum(n, 1e-12)

    return pl.pallas_call(
        body,
        grid=(N_NODES // blk,),
        in_specs=[pl.BlockSpec((blk, D), lambda i: (i, 0))],
        out_specs=pl.BlockSpec((blk, D), lambda i: (i, 0)),
        out_shape=jax.ShapeDtypeStruct((N_NODES, D), jnp.float32),
    )(a)


def kernel(x, edge):
    src = edge[0].astype(jnp.int32)
    dst = edge[2].astype(jnp.int32)

    locs = []
    for h in range(NC):
        l = dst - h * HALF
        locs.append(jnp.where((l >= 0) & (l < HALF), l, HALF))
    dst_f = jnp.concatenate(locs, axis=0)

    xf = x.reshape(N_NODES, NS, LANES).transpose(1, 0, 2).reshape(
        NS * N_NODES, LANES)

    agg = _sc_aggregate(xf, src, dst_f)
    return _tc_finish(agg)


# speedup vs baseline: 1.8370x; 1.0354x over previous
import functools

import jax
import jax.numpy as jnp
from jax import lax
from jax.experimental import pallas as pl
from jax.experimental.pallas import tpu as pltpu
from jax.experimental.pallas import tpu_sc as plsc

N_NODES = 10000
D = 256
E = 160000

NC = 2
NS = 16
LANES = 16

HALF = N_NODES // NC
ACC_R = HALF + 8
SUB = 128
NSUB = 10
CHUNK = SUB * NSUB
NCHUNK = E // CHUNK


def _sc_aggregate(x, src_f, dst_f):
    mesh = plsc.VectorSubcoreMesh(core_axis_name="c", subcore_axis_name="s")

    @functools.partial(
        pl.kernel,
        out_type=jax.ShapeDtypeStruct((N_NODES, D), jnp.float32),
        mesh=mesh,
        compiler_params=pltpu.CompilerParams(use_tc_tiling_on_sc=False),
        scratch_types=[
            pltpu.VMEM((2, CHUNK), jnp.int32),
            pltpu.VMEM((2, CHUNK), jnp.int32),
            pltpu.VMEM((2, SUB, LANES), jnp.float32),
            pltpu.VMEM((ACC_R, LANES), jnp.float32),
            pltpu.SemaphoreType.DMA,
            pltpu.SemaphoreType.DMA,
        ],
    )
    def agg(x_hbm, src_hbm, dst_hbm, out_hbm, src_v, dst_v, rows_v, acc_v,
            isem, gsem):
        h = lax.axis_index("c")
        q = lax.axis_index("s")
        qc = q * LANES

        def idx_copies(k, p):
            kb = k * CHUNK
            a = pltpu.make_async_copy(
                src_hbm.at[pl.ds(kb, CHUNK)], src_v.at[p], isem)
            b = pltpu.make_async_copy(
                dst_hbm.at[pl.ds(h * E + kb, CHUNK)], dst_v.at[p], isem)
            return a, b

        def gather_copy(p, m, mp):
            return pltpu.make_async_copy(
                x_hbm.at[src_v.at[p, pl.ds(m * SUB, SUB)]], rows_v.at[mp],
                gsem)

        for cp in idx_copies(0, 0):
            cp.start()

        def zero(r, carry):
            acc_v[r, :] = jnp.zeros((LANES,), jnp.float32)
            return carry

        lax.fori_loop(0, ACC_R, zero, 0)

        qoff = jnp.full((LANES,), q * N_NODES, jnp.int32)

        def super_chunk(k, carry):
            p = lax.rem(k, 2)
            for cp in idx_copies(k, p):
                cp.wait()

            @pl.when(k < NCHUNK - 1)
            def _prefetch_idx():
                for cp in idx_copies(k + 1, 1 - p):
                    cp.start()

            def offs(i, carry2):
                src_v[p, pl.ds(i * LANES, LANES)] = (
                    src_v[p, pl.ds(i * LANES, LANES)] + qoff)
                return carry2

            lax.fori_loop(0, CHUNK // LANES, offs, 0)

            gather_copy(p, 0, 0).start()

            def sub_chunk(m, carry2):
                mp = lax.rem(m, 2)
                gather_copy(p, m, mp).wait()

                @pl.when(m < NSUB - 1)
                def _prefetch_rows():
                    gather_copy(p, m + 1, 1 - mp).start()

                mb = m * SUB
                for g in range(SUB // LANES):
                    dvec = dst_v[p, pl.ds(mb + g * LANES, LANES)]
                    rows = [rows_v[mp, g * LANES + l, :]
                            for l in range(LANES)]
                    for l in range(LANES):
                        d0 = dvec[l]
                        plsc.addupdate(acc_v.at[d0], rows[l])
                return carry2

            lax.fori_loop(0, NSUB, sub_chunk, 0)
            return carry

        lax.fori_loop(0, NCHUNK, super_chunk, 0)

        pltpu.sync_copy(
            acc_v.at[pl.ds(0, HALF)],
            out_hbm.at[pl.ds(h * HALF, HALF), pl.ds(qc, LANES)])

    return agg(x, src_f, dst_f)


def _tc_finish(a):
    blk = 1000

    def body(a_ref, o_ref):
        v = jnp.maximum(a_ref[...], 0.0)
        n = jnp.sqrt(jnp.sum(v * v, axis=1, keepdims=True))
        o_ref[...] = v / jnp.maximum(n, 1e-12)

    return pl.pallas_call(
        body,
        grid=(N_NODES // blk,),
        in_specs=[pl.BlockSpec((blk, D), lambda i: (i, 0))],
        out_specs=pl.BlockSpec((blk, D), lambda i: (i, 0)),
        out_shape=jax.ShapeDtypeStruct((N_NODES, D), jnp.float32),
    )(a)


def kernel(x, edge):
    src = edge[0].astype(jnp.int32)
    dst = edge[2].astype(jnp.int32)

    locs = []
    for h in range(NC):
        l = dst - h * HALF
        locs.append(jnp.where((l >= 0) & (l < HALF), l, HALF))
    dst_f = jnp.concatenate(locs, axis=0)

    xf = x.reshape(N_NODES, NS, LANES).transpose(1, 0, 2).reshape(
        NS * N_NODES, LANES)

    agg = _sc_aggregate(xf, src, dst_f)
    return _tc_finish(agg)


# speedup vs baseline: 2.8947x; 1.5758x over previous
import functools

import jax
import jax.numpy as jnp
from jax import lax
from jax.experimental import pallas as pl
from jax.experimental.pallas import tpu as pltpu
from jax.experimental.pallas import tpu_sc as plsc

N_NODES = 10000
D = 256
E = 160000

NC = 2
NS = 16
LANES = 16

HALF = N_NODES // NC
ACC_R = HALF + 8
SUB = 128
NSUB = 10
CHUNK = SUB * NSUB
NCHUNK = E // CHUNK


def _sc_aggregate(x, src_f, dst_f):
    mesh = plsc.VectorSubcoreMesh(core_axis_name="c", subcore_axis_name="s")

    @functools.partial(
        pl.kernel,
        out_type=jax.ShapeDtypeStruct((N_NODES, D), jnp.float32),
        mesh=mesh,
        compiler_params=pltpu.CompilerParams(use_tc_tiling_on_sc=False),
        scratch_types=[
            pltpu.VMEM((2, CHUNK), jnp.int32),
            pltpu.VMEM((2, CHUNK), jnp.int32),
            pltpu.VMEM((4, SUB, LANES), jnp.float32),
            pltpu.VMEM((ACC_R, LANES), jnp.float32),
            pltpu.SemaphoreType.DMA,
            pltpu.SemaphoreType.DMA,
        ],
    )
    def agg(x_hbm, src_hbm, dst_hbm, out_hbm, src_v, dst_v, rows_v, acc_v,
            isem, gsem):
        h = lax.axis_index("c")
        q = lax.axis_index("s")
        qc = q * LANES

        def idx_copies(k, p):
            kb = k * CHUNK
            a = pltpu.make_async_copy(
                src_hbm.at[pl.ds(kb, CHUNK)], src_v.at[p], isem)
            b = pltpu.make_async_copy(
                dst_hbm.at[pl.ds(h * E + kb, CHUNK)], dst_v.at[p], isem)
            return a, b

        def gather_copy(p, m, mp):
            return pltpu.make_async_copy(
                x_hbm.at[src_v.at[p, pl.ds(m * SUB, SUB)]], rows_v.at[mp],
                gsem)

        for cp in idx_copies(0, 0):
            cp.start()

        def zero(r, carry):
            acc_v[r, :] = jnp.zeros((LANES,), jnp.float32)
            return carry

        lax.fori_loop(0, ACC_R, zero, 0)

        qoff = jnp.full((LANES,), q * N_NODES, jnp.int32)

        def super_chunk(k, carry):
            p = lax.rem(k, 2)
            for cp in idx_copies(k, p):
                cp.wait()

            @pl.when(k < NCHUNK - 1)
            def _prefetch_idx():
                for cp in idx_copies(k + 1, 1 - p):
                    cp.start()

            def offs(i, carry2):
                src_v[p, pl.ds(i * LANES, LANES)] = (
                    src_v[p, pl.ds(i * LANES, LANES)] + qoff)
                return carry2

            lax.fori_loop(0, CHUNK // LANES, offs, 0)

            for mm in range(3):
                gather_copy(p, mm, mm).start()

            def sub_chunk(m, carry2):
                mp = lax.rem(m, 4)
                gather_copy(p, m, mp).wait()

                @pl.when(m < NSUB - 3)
                def _prefetch_rows():
                    gather_copy(p, m + 3, lax.rem(m + 3, 4)).start()

                mb = m * SUB
                for g in range(SUB // LANES):
                    dvec = dst_v[p, pl.ds(mb + g * LANES, LANES)]
                    rows = [rows_v[mp, g * LANES + l, :]
                            for l in range(LANES)]
                    for l in range(LANES):
                        d0 = dvec[l]
                        plsc.addupdate(acc_v.at[d0], rows[l])
                return carry2

            lax.fori_loop(0, NSUB, sub_chunk, 0)
            return carry

        lax.fori_loop(0, NCHUNK, super_chunk, 0)

        pltpu.sync_copy(
            acc_v.at[pl.ds(0, HALF)],
            out_hbm.at[pl.ds(h * HALF, HALF), pl.ds(qc, LANES)])

    return agg(x, src_f, dst_f)


def _tc_finish(a):
    blk = 1000

    def body(a_ref, o_ref):
        v = jnp.maximum(a_ref[...], 0.0)
        n = jnp.sqrt(jnp.sum(v * v, axis=1, keepdims=True))
        o_ref[...] = v / jnp.maximum(n, 1e-12)

    return pl.pallas_call(
        body,
        grid=(N_NODES // blk,),
        in_specs=[pl.BlockSpec((blk, D), lambda i: (i, 0))],
        out_specs=pl.BlockSpec((blk, D), lambda i: (i, 0)),
        out_shape=jax.ShapeDtypeStruct((N_NODES, D), jnp.float32),
    )(a)


def kernel(x, edge):
    src = edge[0].astype(jnp.int32)
    dst = edge[2].astype(jnp.int32)

    locs = []
    for h in range(NC):
        l = dst - h * HALF
        locs.append(jnp.where((l >= 0) & (l < HALF), l, HALF))
    dst_f = jnp.concatenate(locs, axis=0)

    xf = x.reshape(N_NODES, NS, LANES).transpose(1, 0, 2).reshape(
        NS * N_NODES, LANES)

    agg = _sc_aggregate(xf, src, dst_f)
    return _tc_finish(agg)


# speedup vs baseline: 3.9698x; 1.3714x over previous
import functools

import jax
import jax.numpy as jnp
from jax import lax
from jax.experimental import pallas as pl
from jax.experimental.pallas import tpu as pltpu
from jax.experimental.pallas import tpu_sc as plsc

N_NODES = 10000
D = 256
E = 160000

NC = 2
NS = 16
LANES = 16

HALF = N_NODES // NC
ACC_R = HALF + 8
SUB = 128
NSUB = 10
CHUNK = SUB * NSUB
NCHUNK = E // CHUNK
NSUBG = E // SUB
RING = 6
LOOK = RING - 1


def _sc_aggregate(x, src_f, dst_f):
    mesh = plsc.VectorSubcoreMesh(core_axis_name="c", subcore_axis_name="s")

    @functools.partial(
        pl.kernel,
        out_type=jax.ShapeDtypeStruct((N_NODES, D), jnp.float32),
        mesh=mesh,
        compiler_params=pltpu.CompilerParams(use_tc_tiling_on_sc=False),
        scratch_types=[
            pltpu.VMEM((3, CHUNK), jnp.int32),
            pltpu.VMEM((3, CHUNK), jnp.int32),
            pltpu.VMEM((RING, SUB, LANES), jnp.float32),
            pltpu.VMEM((ACC_R, LANES), jnp.float32),
            pltpu.SemaphoreType.DMA,
            pltpu.SemaphoreType.DMA,
        ],
    )
    def agg(x_hbm, src_hbm, dst_hbm, out_hbm, src_v, dst_v, rows_v, acc_v,
            isem, gsem):
        h = lax.axis_index("c")
        q = lax.axis_index("s")
        qc = q * LANES

        def idx_copies(k, p):
            kb = k * CHUNK
            a = pltpu.make_async_copy(
                src_hbm.at[pl.ds(kb, CHUNK)], src_v.at[p], isem)
            b = pltpu.make_async_copy(
                dst_hbm.at[pl.ds(h * E + kb, CHUNK)], dst_v.at[p], isem)
            return a, b

        def gather_copy(p, m, mp):
            return pltpu.make_async_copy(
                x_hbm.at[src_v.at[p, pl.ds(m * SUB, SUB)]], rows_v.at[mp],
                gsem)

        qoff = jnp.full((LANES,), q * N_NODES, jnp.int32)

        def offs(b):
            def body(i, carry2):
                src_v[b, pl.ds(i * LANES, LANES)] = (
                    src_v[b, pl.ds(i * LANES, LANES)] + qoff)
                return carry2

            lax.fori_loop(0, CHUNK // LANES, body, 0)

        def gather_sub(n):
            b = lax.rem(lax.div(n, NSUB), 3)
            m = lax.rem(n, NSUB)
            return gather_copy(b, m, lax.rem(n, RING))

        for cp in idx_copies(0, 0):
            cp.start()

        def zero(r, carry):
            acc_v[r, :] = jnp.zeros((LANES,), jnp.float32)
            return carry

        lax.fori_loop(0, ACC_R, zero, 0)

        for cp in idx_copies(0, 0):
            cp.wait()
        offs(0)
        for cp in idx_copies(1, 1):
            cp.start()
        for nn in range(LOOK):
            gather_sub(jnp.int32(nn)).start()

        def sub_chunk(n, carry):
            k = lax.div(n, NSUB)
            m = lax.rem(n, NSUB)

            @pl.when((m == 0) & (k < NCHUNK - 1))
            def _stage_next():
                for cp in idx_copies(k + 1, lax.rem(k + 1, 3)):
                    cp.wait()
                offs(lax.rem(k + 1, 3))

                @pl.when(k < NCHUNK - 2)
                def _stage_next2():
                    for cp in idx_copies(k + 2, lax.rem(k + 2, 3)):
                        cp.start()

            mp = lax.rem(n, RING)
            gather_sub(n).wait()

            @pl.when(n < NSUBG - LOOK)
            def _prefetch_rows():
                gather_sub(n + LOOK).start()

            kb = lax.rem(k, 3)
            mb = m * SUB
            for g in range(SUB // LANES):
                dvec = dst_v[kb, pl.ds(mb + g * LANES, LANES)]
                rows = [rows_v[mp, g * LANES + l, :] for l in range(LANES)]
                for l in range(LANES):
                    d0 = dvec[l]
                    plsc.addupdate(acc_v.at[d0], rows[l])
            return carry

        lax.fori_loop(0, NSUBG, sub_chunk, 0)

        pltpu.sync_copy(
            acc_v.at[pl.ds(0, HALF)],
            out_hbm.at[pl.ds(h * HALF, HALF), pl.ds(qc, LANES)])

    return agg(x, src_f, dst_f)


def _tc_finish(a):
    blk = 1000

    def body(a_ref, o_ref):
        v = jnp.maximum(a_ref[...], 0.0)
        n = jnp.sqrt(jnp.sum(v * v, axis=1, keepdims=True))
        o_ref[...] = v / jnp.maximum(n, 1e-12)

    return pl.pallas_call(
        body,
        grid=(N_NODES // blk,),
        in_specs=[pl.BlockSpec((blk, D), lambda i: (i, 0))],
        out_specs=pl.BlockSpec((blk, D), lambda i: (i, 0)),
        out_shape=jax.ShapeDtypeStruct((N_NODES, D), jnp.float32),
    )(a)


def kernel(x, edge):
    src = edge[0].astype(jnp.int32)
    dst = edge[2].astype(jnp.int32)

    locs = []
    for h in range(NC):
        l = dst - h * HALF
        locs.append(jnp.where((l >= 0) & (l < HALF), l, HALF))
    dst_f = jnp.concatenate(locs, axis=0)

    xf = x.reshape(N_NODES, NS, LANES).transpose(1, 0, 2).reshape(
        NS * N_NODES, LANES)

    agg = _sc_aggregate(xf, src, dst_f)
    return _tc_finish(agg)


# speedup vs baseline: 4.2825x; 1.0788x over previous
import functools

import jax
import jax.numpy as jnp
from jax import lax
from jax.experimental import pallas as pl
from jax.experimental.pallas import tpu as pltpu
from jax.experimental.pallas import tpu_sc as plsc

N_NODES = 10000
D = 256
E = 160000

NC = 2
NS = 16
LANES = 16

HALF = N_NODES // NC
ACC_R = HALF + 8
SUB = 128
NSUB = 10
CHUNK = SUB * NSUB
NCHUNK = E // CHUNK
NSUBG = E // SUB
RING = 10
LOOK = RING - 1


def _sc_aggregate(x, src_f, dst_f):
    mesh = plsc.VectorSubcoreMesh(core_axis_name="c", subcore_axis_name="s")

    @functools.partial(
        pl.kernel,
        out_type=jax.ShapeDtypeStruct((N_NODES, D), jnp.float32),
        mesh=mesh,
        compiler_params=pltpu.CompilerParams(use_tc_tiling_on_sc=False),
        scratch_types=[
            pltpu.VMEM((3, CHUNK), jnp.int32),
            pltpu.VMEM((3, CHUNK), jnp.int32),
            pltpu.VMEM((RING, SUB, LANES), jnp.float32),
            pltpu.VMEM((ACC_R, LANES), jnp.float32),
            pltpu.SemaphoreType.DMA,
            pltpu.SemaphoreType.DMA,
        ],
    )
    def agg(x_hbm, src_hbm, dst_hbm, out_hbm, src_v, dst_v, rows_v, acc_v,
            isem, gsem):
        h = lax.axis_index("c")
        q = lax.axis_index("s")
        qc = q * LANES

        def idx_copies(k, p):
            kb = k * CHUNK
            a = pltpu.make_async_copy(
                src_hbm.at[pl.ds(kb, CHUNK)], src_v.at[p], isem)
            b = pltpu.make_async_copy(
                dst_hbm.at[pl.ds(h * E + kb, CHUNK)], dst_v.at[p], isem)
            return a, b

        def gather_copy(p, m, mp):
            return pltpu.make_async_copy(
                x_hbm.at[src_v.at[p, pl.ds(m * SUB, SUB)]], rows_v.at[mp],
                gsem)

        qoff = jnp.full((LANES,), q * N_NODES, jnp.int32)

        def offs(b):
            def body(i, carry2):
                src_v[b, pl.ds(i * LANES, LANES)] = (
                    src_v[b, pl.ds(i * LANES, LANES)] + qoff)
                return carry2

            lax.fori_loop(0, CHUNK // LANES, body, 0)

        def gather_sub(n):
            b = lax.rem(lax.div(n, NSUB), 3)
            m = lax.rem(n, NSUB)
            return gather_copy(b, m, lax.rem(n, RING))

        for cp in idx_copies(0, 0):
            cp.start()

        def zero(r, carry):
            acc_v[r, :] = jnp.zeros((LANES,), jnp.float32)
            return carry

        lax.fori_loop(0, ACC_R, zero, 0)

        for cp in idx_copies(0, 0):
            cp.wait()
        offs(0)
        for cp in idx_copies(1, 1):
            cp.start()
        for nn in range(LOOK):
            gather_sub(jnp.int32(nn)).start()

        def sub_chunk(n, carry):
            k = lax.div(n, NSUB)
            m = lax.rem(n, NSUB)

            @pl.when((m == 0) & (k < NCHUNK - 1))
            def _stage_next():
                for cp in idx_copies(k + 1, lax.rem(k + 1, 3)):
                    cp.wait()
                offs(lax.rem(k + 1, 3))

                @pl.when(k < NCHUNK - 2)
                def _stage_next2():
                    for cp in idx_copies(k + 2, lax.rem(k + 2, 3)):
                        cp.start()

            mp = lax.rem(n, RING)
            gather_sub(n).wait()

            @pl.when(n < NSUBG - LOOK)
            def _prefetch_rows():
                gather_sub(n + LOOK).start()

            kb = lax.rem(k, 3)
            mb = m * SUB
            for g in range(SUB // LANES):
                dvec = dst_v[kb, pl.ds(mb + g * LANES, LANES)]
                rows = [rows_v[mp, g * LANES + l, :] for l in range(LANES)]
                for l in range(LANES):
                    d0 = dvec[l]
                    plsc.addupdate(acc_v.at[d0], rows[l])
            return carry

        lax.fori_loop(0, NSUBG, sub_chunk, 0)

        pltpu.sync_copy(
            acc_v.at[pl.ds(0, HALF)],
            out_hbm.at[pl.ds(h * HALF, HALF), pl.ds(qc, LANES)])

    return agg(x, src_f, dst_f)


def _tc_finish(a):
    blk = 1000

    def body(a_ref, o_ref):
        v = jnp.maximum(a_ref[...], 0.0)
        n = jnp.sqrt(jnp.sum(v * v, axis=1, keepdims=True))
        o_ref[...] = v / jnp.maximum(n, 1e-12)

    return pl.pallas_call(
        body,
        grid=(N_NODES // blk,),
        in_specs=[pl.BlockSpec((blk, D), lambda i: (i, 0))],
        out_specs=pl.BlockSpec((blk, D), lambda i: (i, 0)),
        out_shape=jax.ShapeDtypeStruct((N_NODES, D), jnp.float32),
    )(a)


def kernel(x, edge):
    src = edge[0].astype(jnp.int32)
    dst = edge[2].astype(jnp.int32)

    locs = []
    for h in range(NC):
        l = dst - h * HALF
        locs.append(jnp.where((l >= 0) & (l < HALF), l, HALF))
    dst_f = jnp.concatenate(locs, axis=0)

    xf = x.reshape(N_NODES, NS, LANES).transpose(1, 0, 2).reshape(
        NS * N_NODES, LANES)

    agg = _sc_aggregate(xf, src, dst_f)
    return _tc_finish(agg)


# speedup vs baseline: 4.3722x; 1.0209x over previous
import functools

import jax
import jax.numpy as jnp
from jax import lax
from jax.experimental import pallas as pl
from jax.experimental.pallas import tpu as pltpu
from jax.experimental.pallas import tpu_sc as plsc

N_NODES = 10000
D = 256
E = 160000

NC = 2
NS = 16
LANES = 16

HALF = N_NODES // NC
ACC_R = HALF + 8
SUB = 128
NSUB = 10
CHUNK = SUB * NSUB
NCHUNK = E // CHUNK
NSUBG = E // SUB
RING = 12
LOOK = 10


def _sc_aggregate(x, src_f, dst_f):
    mesh = plsc.VectorSubcoreMesh(core_axis_name="c", subcore_axis_name="s")

    @functools.partial(
        pl.kernel,
        out_type=jax.ShapeDtypeStruct((N_NODES, D), jnp.float32),
        mesh=mesh,
        compiler_params=pltpu.CompilerParams(use_tc_tiling_on_sc=False),
        scratch_types=[
            pltpu.VMEM((3, CHUNK), jnp.int32),
            pltpu.VMEM((3, CHUNK), jnp.int32),
            pltpu.VMEM((RING, SUB, LANES), jnp.float32),
            pltpu.VMEM((ACC_R, LANES), jnp.float32),
            pltpu.SemaphoreType.DMA,
            pltpu.SemaphoreType.DMA,
        ],
    )
    def agg(x_hbm, src_hbm, dst_hbm, out_hbm, src_v, dst_v, rows_v, acc_v,
            isem, gsem):
        h = lax.axis_index("c")
        q = lax.axis_index("s")
        qc = q * LANES

        def idx_copies(k, p):
            kb = k * CHUNK
            a = pltpu.make_async_copy(
                src_hbm.at[pl.ds(kb, CHUNK)], src_v.at[p], isem)
            b = pltpu.make_async_copy(
                dst_hbm.at[pl.ds(h * E + kb, CHUNK)], dst_v.at[p], isem)
            return a, b

        def gather_copy(p, m, mp):
            return pltpu.make_async_copy(
                x_hbm.at[src_v.at[p, pl.ds(m * SUB, SUB)]], rows_v.at[mp],
                gsem)

        qoff = jnp.full((LANES,), q * N_NODES, jnp.int32)

        def offs(b):
            def body(i, carry2):
                src_v[b, pl.ds(i * LANES, LANES)] = (
                    src_v[b, pl.ds(i * LANES, LANES)] + qoff)
                return carry2

            lax.fori_loop(0, CHUNK // LANES, body, 0)

        def gather_sub(n):
            b = lax.rem(lax.div(n, NSUB), 3)
            m = lax.rem(n, NSUB)
            return gather_copy(b, m, lax.rem(n, RING))

        for cp in idx_copies(0, 0):
            cp.start()

        def zero(r, carry):
            acc_v[r, :] = jnp.zeros((LANES,), jnp.float32)
            return carry

        lax.fori_loop(0, ACC_R, zero, 0)

        for cp in idx_copies(0, 0):
            cp.wait()
        offs(0)
        for cp in idx_copies(1, 1):
            cp.start()
        for nn in range(LOOK):
            gather_sub(jnp.int32(nn)).start()

        def sub_chunk(n, carry):
            k = lax.div(n, NSUB)
            m = lax.rem(n, NSUB)

            @pl.when((m == 0) & (k < NCHUNK - 1))
            def _stage_next():
                for cp in idx_copies(k + 1, lax.rem(k + 1, 3)):
                    cp.wait()
                offs(lax.rem(k + 1, 3))

                @pl.when(k < NCHUNK - 2)
                def _stage_next2():
                    for cp in idx_copies(k + 2, lax.rem(k + 2, 3)):
                        cp.start()

            mp = lax.rem(n, RING)
            gather_sub(n).wait()

            @pl.when(n < NSUBG - LOOK)
            def _prefetch_rows():
                gather_sub(n + LOOK).start()

            kb = lax.rem(k, 3)
            mb = m * SUB
            for g in range(SUB // LANES):
                dvec = dst_v[kb, pl.ds(mb + g * LANES, LANES)]
                rows = [rows_v[mp, g * LANES + l, :] for l in range(LANES)]
                for l in range(LANES):
                    d0 = dvec[l]
                    plsc.addupdate(acc_v.at[d0], rows[l])
            return carry

        lax.fori_loop(0, NSUBG, sub_chunk, 0)

        pltpu.sync_copy(
            acc_v.at[pl.ds(0, HALF)],
            out_hbm.at[pl.ds(h * HALF, HALF), pl.ds(qc, LANES)])

    return agg(x, src_f, dst_f)


def _tc_finish(a):
    blk = 1000

    def body(a_ref, o_ref):
        v = jnp.maximum(a_ref[...], 0.0)
        n = jnp.sqrt(jnp.sum(v * v, axis=1, keepdims=True))
        o_ref[...] = v / jnp.maximum(n, 1e-12)

    return pl.pallas_call(
        body,
        grid=(N_NODES // blk,),
        in_specs=[pl.BlockSpec((blk, D), lambda i: (i, 0))],
        out_specs=pl.BlockSpec((blk, D), lambda i: (i, 0)),
        out_shape=jax.ShapeDtypeStruct((N_NODES, D), jnp.float32),
    )(a)


def kernel(x, edge):
    src = edge[0].astype(jnp.int32)
    dst = edge[2].astype(jnp.int32)

    locs = []
    for h in range(NC):
        l = dst - h * HALF
        locs.append(jnp.where((l >= 0) & (l < HALF), l, HALF))
    dst_f = jnp.concatenate(locs, axis=0)

    xf = x.reshape(N_NODES, NS, LANES).transpose(1, 0, 2).reshape(
        NS * N_NODES, LANES)

    agg = _sc_aggregate(xf, src, dst_f)
    return _tc_finish(agg)
